# Initial kernel scaffold; baseline (speedup 1.0000x reference)
import functools

import jax
import jax.numpy as jnp
from jax import lax
from jax.experimental import pallas as pl
from jax.experimental.pallas import tpu as pltpu
from jax.experimental.pallas import tpu_sc as plsc

N = 10000
E = 320000
D = 128
R = 16

NC = 2
NS = 16
NW = NC * NS
EPW = E // NW
CHUNK = 80
NCHUNK = EPW // CHUNK
ROWS_PER_SUB = N // NS

NODE_BLK = 1000
EDGE_BLK = 4000


def _dense_body(x_ref, w_ref, b_ref, o_ref):
    o_ref[...] = (
        jnp.dot(x_ref[...], w_ref[...], preferred_element_type=jnp.float32)
        + b_ref[...]
    )


def _matmul_body(x_ref, w_ref, o_ref):
    o_ref[...] = jnp.dot(x_ref[...], w_ref[...],
                         preferred_element_type=jnp.float32)


def _make_edge_kernel():
    mesh = plsc.VectorSubcoreMesh(core_axis_name="c", subcore_axis_name="s")

    @functools.partial(
        pl.kernel,
        mesh=mesh,
        out_type=jax.ShapeDtypeStruct((NC, N, D), jnp.float32),
        scratch_types=[
            pltpu.VMEM((CHUNK,), jnp.int32),
            pltpu.VMEM((CHUNK,), jnp.int32),
            pltpu.VMEM((CHUNK, D), jnp.float32),
            pltpu.VMEM((CHUNK, D), jnp.float32),
            pltpu.VMEM_SHARED((N, D), jnp.float32),
            pltpu.SemaphoreType.DMA,
        ],
    )
    def edge_kernel(g_hbm, idxj_hbm, idxi_hbm, xj_hbm, zeros_hbm, out_hbm,
                    idxj_v, idxi_v, g_v, rows_v, acc_sh, sem):
        cid = lax.axis_index("c")
        sid = lax.axis_index("s")
        wid = sid * NC + cid

        zbase = sid * ROWS_PER_SUB
        pltpu.sync_copy(zeros_hbm.at[pl.ds(zbase, ROWS_PER_SUB), :],
                        acc_sh.at[pl.ds(zbase, ROWS_PER_SUB), :])
        plsc.subcore_barrier()

        def chunk_body(ci, carry):
            base = wid * EPW + ci * CHUNK
            pltpu.sync_copy(idxj_hbm.at[pl.ds(base, CHUNK)], idxj_v)
            pltpu.sync_copy(idxi_hbm.at[pl.ds(base, CHUNK)], idxi_v)
            pltpu.sync_copy(g_hbm.at[pl.ds(base, CHUNK), :], g_v)
            pltpu.async_copy(xj_hbm.at[idxj_v], rows_v, sem).wait()

            def row_body(i, c2):
                for d in range(D // 16):
                    s = pl.ds(d * 16, 16)
                    rows_v[i, s] = rows_v[i, s] * g_v[i, s]
                return c2

            lax.fori_loop(0, CHUNK, row_body, 0)
            pltpu.sync_copy(rows_v, acc_sh.at[idxi_v], add=True)
            return carry

        lax.fori_loop(0, NCHUNK, chunk_body, 0)
        plsc.subcore_barrier()
        pltpu.sync_copy(acc_sh.at[pl.ds(zbase, ROWS_PER_SUB), :],
                        out_hbm.at[cid, pl.ds(zbase, ROWS_PER_SUB), :])

    return edge_kernel


_edge_kernel = _make_edge_kernel()


def _node_stack_body(x_ref, p0_ref, p1_ref, wi_ref, bi_ref,
                     rw1_ref, rb1_ref, rw2_ref, rb2_ref,
                     wo_ref, bo_ref, sc_ref,
                     fw1_ref, fb1_ref, fw2_ref, fb2_ref, o_ref):
    x = x_ref[...]
    msg = (jnp.dot(x, wi_ref[...], preferred_element_type=jnp.float32)
           + bi_ref[...] + p0_ref[...] + p1_ref[...])
    for k in range(rw1_ref.shape[0]):
        h = jnp.dot(msg, rw1_ref[k], preferred_element_type=jnp.float32)
        h = h + rb1_ref[k]
        msg = msg + jnp.dot(h, rw2_ref[k],
                            preferred_element_type=jnp.float32) + rb2_ref[k]
    y = (sc_ref[...] * x
         + jnp.dot(msg, wo_ref[...], preferred_element_type=jnp.float32)
         + bo_ref[...])
    for k in range(fw1_ref.shape[0]):
        h = jnp.dot(y, fw1_ref[k], preferred_element_type=jnp.float32)
        h = h + fb1_ref[k]
        y = y + jnp.dot(h, fw2_ref[k],
                        preferred_element_type=jnp.float32) + fb2_ref[k]
    o_ref[...] = y


def _full(shape):
    return pl.BlockSpec(shape, lambda i: tuple(0 for _ in shape))


def kernel(features, descriptors, idx_i, idx_j, W_r2a, W_i, b_i, W_j, b_j,
           res_ij_W1, res_ij_b1, res_ij_W2, res_ij_b2, W_out, b_out, scaling,
           res_f_W1, res_f_b1, res_f_W2, res_f_b2):
    idx_i = idx_i.astype(jnp.int32)
    idx_j = idx_j.astype(jnp.int32)

    xj_all = pl.pallas_call(
        _dense_body,
        grid=(N // NODE_BLK,),
        in_specs=[pl.BlockSpec((NODE_BLK, D), lambda i: (i, 0)),
                  _full((D, D)),
                  _full((1, D))],
        out_specs=pl.BlockSpec((NODE_BLK, D), lambda i: (i, 0)),
        out_shape=jax.ShapeDtypeStruct((N, D), jnp.float32),
    )(features, W_j, b_j.reshape(1, D))

    g = pl.pallas_call(
        _matmul_body,
        grid=(E // EDGE_BLK,),
        in_specs=[pl.BlockSpec((EDGE_BLK, R), lambda i: (i, 0)),
                  _full((R, D))],
        out_specs=pl.BlockSpec((EDGE_BLK, D), lambda i: (i, 0)),
        out_shape=jax.ShapeDtypeStruct((E, D), jnp.float32),
    )(descriptors, W_r2a)

    zeros = jnp.zeros((N, D), jnp.float32)
    partials = _edge_kernel(g, idx_j, idx_i, xj_all, zeros)

    out = pl.pallas_call(
        _node_stack_body,
        grid=(N // NODE_BLK,),
        in_specs=[pl.BlockSpec((NODE_BLK, D), lambda i: (i, 0)),
                  pl.BlockSpec((NODE_BLK, D), lambda i: (i, 0)),
                  pl.BlockSpec((NODE_BLK, D), lambda i: (i, 0)),
                  _full((D, D)), _full((1, D)),
                  _full((2, D, D)), _full((2, 1, D)),
                  _full((2, D, D)), _full((2, 1, D)),
                  _full((D, D)), _full((1, D)), _full((1, D)),
                  _full((2, D, D)), _full((2, 1, D)),
                  _full((2, D, D)), _full((2, 1, D))],
        out_specs=pl.BlockSpec((NODE_BLK, D), lambda i: (i, 0)),
        out_shape=jax.ShapeDtypeStruct((N, D), jnp.float32),
    )(features, partials[0], partials[1],
      W_i, b_i.reshape(1, D),
      res_ij_W1, res_ij_b1.reshape(2, 1, D),
      res_ij_W2, res_ij_b2.reshape(2, 1, D),
      W_out, b_out.reshape(1, D), scaling.reshape(1, D),
      res_f_W1, res_f_b1.reshape(2, 1, D),
      res_f_W2, res_f_b2.reshape(2, 1, D))
    return out


# speedup vs baseline: 2.4916x; 2.4916x over previous
import functools

import jax
import jax.numpy as jnp
from jax import lax
from jax.experimental import pallas as pl
from jax.experimental.pallas import tpu as pltpu
from jax.experimental.pallas import tpu_sc as plsc

N = 10000
E = 320000
D = 128
R = 16

NC = 2
NS = 16
NW = NC * NS
EPW = E // NW
CHUNK = 80
NCHUNK = EPW // CHUNK
NPAD = 10240
ROWS_PER_SUB = NPAD // NS

NODE_BLK = 1000
EDGE_BLK = 4000


def _dense_body(x_ref, w_ref, b_ref, o_ref):
    o_ref[...] = (
        jnp.dot(x_ref[...], w_ref[...], preferred_element_type=jnp.float32)
        + b_ref[...]
    )


def _matmul_body(x_ref, w_ref, o_ref):
    o_ref[...] = jnp.dot(x_ref[...], w_ref[...],
                         preferred_element_type=jnp.float32)


def _make_edge_kernel():
    mesh = plsc.VectorSubcoreMesh(core_axis_name="c", subcore_axis_name="s")

    @functools.partial(
        pl.kernel,
        mesh=mesh,
        out_type=jax.ShapeDtypeStruct((NC, NPAD, D), jnp.float32),
        scratch_types=[
            pltpu.VMEM((CHUNK,), jnp.int32),
            pltpu.VMEM((CHUNK,), jnp.int32),
            pltpu.VMEM((CHUNK, D), jnp.float32),
            pltpu.VMEM((CHUNK, D), jnp.float32),
            pltpu.VMEM_SHARED((NPAD, D), jnp.float32),
            pltpu.SemaphoreType.DMA,
        ],
    )
    def edge_kernel(g_hbm, idxj_hbm, idxi_hbm, xj_hbm, zeros_hbm, out_hbm,
                    idxj_v, idxi_v, g_v, rows_v, acc_sh, sem):
        cid = lax.axis_index("c")
        sid = lax.axis_index("s")
        wid = sid * NC + cid

        zbase = sid * ROWS_PER_SUB
        pltpu.sync_copy(zeros_hbm.at[pl.ds(zbase, ROWS_PER_SUB), :],
                        acc_sh.at[pl.ds(zbase, ROWS_PER_SUB), :])
        plsc.subcore_barrier()

        def chunk_body(ci, carry):
            base = wid * EPW + ci * CHUNK
            pltpu.sync_copy(idxj_hbm.at[pl.ds(base, CHUNK)], idxj_v)
            pltpu.sync_copy(idxi_hbm.at[pl.ds(base, CHUNK)], idxi_v)
            pltpu.sync_copy(g_hbm.at[pl.ds(base, CHUNK), :], g_v)
            pltpu.async_copy(xj_hbm.at[idxj_v], rows_v, sem).wait()

            def row_body(i, c2):
                for d in range(D // 16):
                    s = pl.ds(d * 16, 16)
                    rows_v[i, s] = rows_v[i, s] * g_v[i, s]
                return c2

            lax.fori_loop(0, CHUNK, row_body, 0)
            pltpu.sync_copy(rows_v, acc_sh.at[idxi_v], add=True)
            return carry

        lax.fori_loop(0, NCHUNK, chunk_body, 0)
        plsc.subcore_barrier()
        pltpu.sync_copy(acc_sh.at[pl.ds(zbase, ROWS_PER_SUB), :],
                        out_hbm.at[cid, pl.ds(zbase, ROWS_PER_SUB), :])

    return edge_kernel


_edge_kernel = _make_edge_kernel()


def _node_stack_body(x_ref, p0_ref, p1_ref, wi_ref, bi_ref,
                     rw1_ref, rb1_ref, rw2_ref, rb2_ref,
                     wo_ref, bo_ref, sc_ref,
                     fw1_ref, fb1_ref, fw2_ref, fb2_ref, o_ref):
    x = x_ref[...]
    msg = (jnp.dot(x, wi_ref[...], preferred_element_type=jnp.float32)
           + bi_ref[...] + p0_ref[...] + p1_ref[...])
    for k in range(rw1_ref.shape[0]):
        h = jnp.dot(msg, rw1_ref[k], preferred_element_type=jnp.float32)
        h = h + rb1_ref[k]
        msg = msg + jnp.dot(h, rw2_ref[k],
                            preferred_element_type=jnp.float32) + rb2_ref[k]
    y = (sc_ref[...] * x
         + jnp.dot(msg, wo_ref[...], preferred_element_type=jnp.float32)
         + bo_ref[...])
    for k in range(fw1_ref.shape[0]):
        h = jnp.dot(y, fw1_ref[k], preferred_element_type=jnp.float32)
        h = h + fb1_ref[k]
        y = y + jnp.dot(h, fw2_ref[k],
                        preferred_element_type=jnp.float32) + fb2_ref[k]
    o_ref[...] = y


def _full(shape):
    return pl.BlockSpec(shape, lambda i: tuple(0 for _ in shape))


def kernel(features, descriptors, idx_i, idx_j, W_r2a, W_i, b_i, W_j, b_j,
           res_ij_W1, res_ij_b1, res_ij_W2, res_ij_b2, W_out, b_out, scaling,
           res_f_W1, res_f_b1, res_f_W2, res_f_b2):
    idx_i = idx_i.astype(jnp.int32)
    idx_j = idx_j.astype(jnp.int32)

    xj_all = pl.pallas_call(
        _dense_body,
        grid=(N // NODE_BLK,),
        in_specs=[pl.BlockSpec((NODE_BLK, D), lambda i: (i, 0)),
                  _full((D, D)),
                  _full((1, D))],
        out_specs=pl.BlockSpec((NODE_BLK, D), lambda i: (i, 0)),
        out_shape=jax.ShapeDtypeStruct((N, D), jnp.float32),
    )(features, W_j, b_j.reshape(1, D))

    g = pl.pallas_call(
        _matmul_body,
        grid=(E // EDGE_BLK,),
        in_specs=[pl.BlockSpec((EDGE_BLK, R), lambda i: (i, 0)),
                  _full((R, D))],
        out_specs=pl.BlockSpec((EDGE_BLK, D), lambda i: (i, 0)),
        out_shape=jax.ShapeDtypeStruct((E, D), jnp.float32),
    )(descriptors, W_r2a)

    zeros = jnp.zeros((NPAD, D), jnp.float32)
    partials = _edge_kernel(g, idx_j, idx_i, xj_all, zeros)
    partials = partials[:, :N, :]

    out = pl.pallas_call(
        _node_stack_body,
        grid=(N // NODE_BLK,),
        in_specs=[pl.BlockSpec((NODE_BLK, D), lambda i: (i, 0)),
                  pl.BlockSpec((NODE_BLK, D), lambda i: (i, 0)),
                  pl.BlockSpec((NODE_BLK, D), lambda i: (i, 0)),
                  _full((D, D)), _full((1, D)),
                  _full((2, D, D)), _full((2, 1, D)),
                  _full((2, D, D)), _full((2, 1, D)),
                  _full((D, D)), _full((1, D)), _full((1, D)),
                  _full((2, D, D)), _full((2, 1, D)),
                  _full((2, D, D)), _full((2, 1, D))],
        out_specs=pl.BlockSpec((NODE_BLK, D), lambda i: (i, 0)),
        out_shape=jax.ShapeDtypeStruct((N, D), jnp.float32),
    )(features, partials[0], partials[1],
      W_i, b_i.reshape(1, D),
      res_ij_W1, res_ij_b1.reshape(2, 1, D),
      res_ij_W2, res_ij_b2.reshape(2, 1, D),
      W_out, b_out.reshape(1, D), scaling.reshape(1, D),
      res_f_W1, res_f_b1.reshape(2, 1, D),
      res_f_W2, res_f_b2.reshape(2, 1, D))
    return out


# speedup vs baseline: 3.9446x; 1.5832x over previous
import functools

import jax
import jax.numpy as jnp
from jax import lax
from jax.experimental import pallas as pl
from jax.experimental.pallas import tpu as pltpu
from jax.experimental.pallas import tpu_sc as plsc

N = 10000
E = 320000
D = 128
R = 16

NC = 2
NS = 16
NW = NC * NS
EPW = E // NW
CHUNK = 40
NCHUNK = EPW // CHUNK
NPAD = 10240
ROWS_PER_SUB = NPAD // NS

NODE_BLK = 1000
EDGE_BLK = 4000


def _dense_body(x_ref, w_ref, b_ref, o_ref):
    o_ref[...] = (
        jnp.dot(x_ref[...], w_ref[...], preferred_element_type=jnp.float32)
        + b_ref[...]
    )


def _matmul_body(x_ref, w_ref, o_ref):
    o_ref[...] = jnp.dot(x_ref[...], w_ref[...],
                         preferred_element_type=jnp.float32)


NB = 4


def _make_edge_kernel():
    mesh = plsc.VectorSubcoreMesh(core_axis_name="c", subcore_axis_name="s")

    @functools.partial(
        pl.kernel,
        mesh=mesh,
        out_type=jax.ShapeDtypeStruct((NC, NPAD, D), jnp.float32),
        scratch_types=[
            pltpu.VMEM((NB, CHUNK), jnp.int32),
            pltpu.VMEM((NB, CHUNK), jnp.int32),
            pltpu.VMEM((NB, CHUNK, D), jnp.float32),
            pltpu.VMEM((NB, CHUNK, D), jnp.float32),
            pltpu.VMEM_SHARED((NPAD, D), jnp.float32),
            pltpu.SemaphoreType.DMA((NB,)),
            pltpu.SemaphoreType.DMA((NB,)),
            pltpu.SemaphoreType.DMA((NB,)),
        ],
    )
    def edge_kernel(g_hbm, idxj_hbm, idxi_hbm, xj_hbm, zeros_hbm, out_hbm,
                    idxj_v, idxi_v, g_v, rows_v, acc_sh,
                    sem_in, sem_gat, sem_sc):
        cid = lax.axis_index("c")
        sid = lax.axis_index("s")
        wid = sid * NC + cid

        zbase = sid * ROWS_PER_SUB
        pltpu.sync_copy(zeros_hbm.at[pl.ds(zbase, ROWS_PER_SUB), :],
                        acc_sh.at[pl.ds(zbase, ROWS_PER_SUB), :])
        plsc.subcore_barrier()

        def in_copies(ci, b):
            base = wid * EPW + ci * CHUNK
            return (
                (idxj_hbm.at[pl.ds(base, CHUNK)], idxj_v.at[b], sem_in.at[b]),
                (idxi_hbm.at[pl.ds(base, CHUNK)], idxi_v.at[b], sem_in.at[b]),
                (g_hbm.at[pl.ds(base, CHUNK), :], g_v.at[b], sem_in.at[b]),
            )

        def in_start(ci, b):
            for args in in_copies(ci, b):
                pltpu.async_copy(*args)

        def in_wait(ci, b):
            for args in in_copies(ci, b):
                pltpu.make_async_copy(*args).wait()

        def gat_start(b):
            pltpu.async_copy(xj_hbm.at[idxj_v.at[b]], rows_v.at[b],
                             sem_gat.at[b])

        def gat_wait(b):
            pltpu.make_async_copy(xj_hbm.at[idxj_v.at[b]], rows_v.at[b],
                                  sem_gat.at[b]).wait()

        def sc_start(b):
            pltpu.async_copy(rows_v.at[b], acc_sh.at[idxi_v.at[b]],
                             sem_sc.at[b], add=True)

        def sc_wait(b):
            pltpu.make_async_copy(rows_v.at[b], acc_sh.at[idxi_v.at[b]],
                                  sem_sc.at[b]).wait()

        def mul(b):
            def row_body(i, c2):
                for d in range(D // 16):
                    s = pl.ds(d * 16, 16)
                    rows_v[b, i, s] = rows_v[b, i, s] * g_v[b, i, s]
                return c2

            lax.fori_loop(0, CHUNK, row_body, 0)

        def process(ci, b, guards_static):
            gat_wait(b)
            if (not guards_static) or ci + 1 < NCHUNK:
                b1 = (b + 1) % NB
                in_wait(ci + 1, b1)
                gat_start(b1)
            mul(b)
            sc_start(b)
            if (not guards_static) or ci + 3 < NCHUNK:
                b3 = (b + 3) % NB
                if (not guards_static) or ci + 3 - NB >= 0:
                    sc_wait(b3)
                in_start(ci + 3, b3)

        in_start(0, 0)
        in_start(1, 1)
        in_start(2, 2)
        in_wait(0, 0)
        gat_start(0)
        for ci in range(NB):
            process(ci, ci % NB, guards_static=True)

        def outer_body(o, carry):
            for b in range(NB):
                process(o * NB + b, b, guards_static=False)
            return carry

        lax.fori_loop(1, NCHUNK // NB - 1, outer_body, 0)

        for ci in range((NCHUNK // NB - 1) * NB, NCHUNK):
            process(ci, ci % NB, guards_static=True)
        for b in range(NB):
            sc_wait(b)

        plsc.subcore_barrier()
        pltpu.sync_copy(acc_sh.at[pl.ds(zbase, ROWS_PER_SUB), :],
                        out_hbm.at[cid, pl.ds(zbase, ROWS_PER_SUB), :])

    return edge_kernel


_edge_kernel = _make_edge_kernel()


def _node_stack_body(x_ref, p0_ref, p1_ref, wi_ref, bi_ref,
                     rw1_ref, rb1_ref, rw2_ref, rb2_ref,
                     wo_ref, bo_ref, sc_ref,
                     fw1_ref, fb1_ref, fw2_ref, fb2_ref, o_ref):
    x = x_ref[...]
    msg = (jnp.dot(x, wi_ref[...], preferred_element_type=jnp.float32)
           + bi_ref[...] + p0_ref[...] + p1_ref[...])
    for k in range(rw1_ref.shape[0]):
        h = jnp.dot(msg, rw1_ref[k], preferred_element_type=jnp.float32)
        h = h + rb1_ref[k]
        msg = msg + jnp.dot(h, rw2_ref[k],
                            preferred_element_type=jnp.float32) + rb2_ref[k]
    y = (sc_ref[...] * x
         + jnp.dot(msg, wo_ref[...], preferred_element_type=jnp.float32)
         + bo_ref[...])
    for k in range(fw1_ref.shape[0]):
        h = jnp.dot(y, fw1_ref[k], preferred_element_type=jnp.float32)
        h = h + fb1_ref[k]
        y = y + jnp.dot(h, fw2_ref[k],
                        preferred_element_type=jnp.float32) + fb2_ref[k]
    o_ref[...] = y


def _full(shape):
    return pl.BlockSpec(shape, lambda i: tuple(0 for _ in shape))


def kernel(features, descriptors, idx_i, idx_j, W_r2a, W_i, b_i, W_j, b_j,
           res_ij_W1, res_ij_b1, res_ij_W2, res_ij_b2, W_out, b_out, scaling,
           res_f_W1, res_f_b1, res_f_W2, res_f_b2):
    idx_i = idx_i.astype(jnp.int32)
    idx_j = idx_j.astype(jnp.int32)

    xj_all = pl.pallas_call(
        _dense_body,
        grid=(N // NODE_BLK,),
        in_specs=[pl.BlockSpec((NODE_BLK, D), lambda i: (i, 0)),
                  _full((D, D)),
                  _full((1, D))],
        out_specs=pl.BlockSpec((NODE_BLK, D), lambda i: (i, 0)),
        out_shape=jax.ShapeDtypeStruct((N, D), jnp.float32),
    )(features, W_j, b_j.reshape(1, D))

    g = pl.pallas_call(
        _matmul_body,
        grid=(E // EDGE_BLK,),
        in_specs=[pl.BlockSpec((EDGE_BLK, R), lambda i: (i, 0)),
                  _full((R, D))],
        out_specs=pl.BlockSpec((EDGE_BLK, D), lambda i: (i, 0)),
        out_shape=jax.ShapeDtypeStruct((E, D), jnp.float32),
    )(descriptors, W_r2a)

    zeros = jnp.zeros((NPAD, D), jnp.float32)
    partials = _edge_kernel(g, idx_j, idx_i, xj_all, zeros)
    partials = partials[:, :N, :]

    out = pl.pallas_call(
        _node_stack_body,
        grid=(N // NODE_BLK,),
        in_specs=[pl.BlockSpec((NODE_BLK, D), lambda i: (i, 0)),
                  pl.BlockSpec((NODE_BLK, D), lambda i: (i, 0)),
                  pl.BlockSpec((NODE_BLK, D), lambda i: (i, 0)),
                  _full((D, D)), _full((1, D)),
                  _full((2, D, D)), _full((2, 1, D)),
                  _full((2, D, D)), _full((2, 1, D)),
                  _full((D, D)), _full((1, D)), _full((1, D)),
                  _full((2, D, D)), _full((2, 1, D)),
                  _full((2, D, D)), _full((2, 1, D))],
        out_specs=pl.BlockSpec((NODE_BLK, D), lambda i: (i, 0)),
        out_shape=jax.ShapeDtypeStruct((N, D), jnp.float32),
    )(features, partials[0], partials[1],
      W_i, b_i.reshape(1, D),
      res_ij_W1, res_ij_b1.reshape(2, 1, D),
      res_ij_W2, res_ij_b2.reshape(2, 1, D),
      W_out, b_out.reshape(1, D), scaling.reshape(1, D),
      res_f_W1, res_f_b1.reshape(2, 1, D),
      res_f_W2, res_f_b2.reshape(2, 1, D))
    return out


# speedup vs baseline: 3.9977x; 1.0134x over previous
import functools

import jax
import jax.numpy as jnp
from jax import lax
from jax.experimental import pallas as pl
from jax.experimental.pallas import tpu as pltpu
from jax.experimental.pallas import tpu_sc as plsc

N = 10000
E = 320000
D = 128
R = 16

NC = 2
NS = 16
NW = NC * NS
EPW = E // NW
NHALF = 2
EPW_H = E // (NHALF * NW)
CHUNK = 40
NCHUNK = EPW_H // CHUNK
NPAD = 10240
ROWS_PER_SUB = NPAD // NS

NODE_BLK = 1000
EDGE_BLK = 4000


def _dense_body(x_ref, w_ref, b_ref, o_ref):
    o_ref[...] = (
        jnp.dot(x_ref[...], w_ref[...], preferred_element_type=jnp.float32)
        + b_ref[...]
    )


def _matmul_body(x_ref, w_ref, o_ref):
    o_ref[...] = jnp.dot(x_ref[...], w_ref[...],
                         preferred_element_type=jnp.float32)


NB = 4


def _make_edge_kernel(base0):
    mesh = plsc.VectorSubcoreMesh(core_axis_name="c", subcore_axis_name="s")

    @functools.partial(
        pl.kernel,
        mesh=mesh,
        out_type=jax.ShapeDtypeStruct((NC, NPAD, D), jnp.float32),
        scratch_types=[
            pltpu.VMEM((NB, CHUNK), jnp.int32),
            pltpu.VMEM((NB, CHUNK), jnp.int32),
            pltpu.VMEM((NB, CHUNK, D), jnp.float32),
            pltpu.VMEM((NB, CHUNK, D), jnp.float32),
            pltpu.VMEM_SHARED((NPAD, D), jnp.float32),
            pltpu.SemaphoreType.DMA((NB,)),
            pltpu.SemaphoreType.DMA((NB,)),
            pltpu.SemaphoreType.DMA((NB,)),
        ],
    )
    def edge_kernel(g_hbm, idxj_hbm, idxi_hbm, xj_hbm, zeros_hbm, out_hbm,
                    idxj_v, idxi_v, g_v, rows_v, acc_sh,
                    sem_in, sem_gat, sem_sc):
        cid = lax.axis_index("c")
        sid = lax.axis_index("s")
        wid = sid * NC + cid

        zbase = sid * ROWS_PER_SUB
        pltpu.sync_copy(zeros_hbm.at[pl.ds(zbase, ROWS_PER_SUB), :],
                        acc_sh.at[pl.ds(zbase, ROWS_PER_SUB), :])
        plsc.subcore_barrier()

        def in_copies(ci, b):
            gbase = wid * EPW_H + ci * CHUNK
            base = base0 + gbase
            return (
                (idxj_hbm.at[pl.ds(base, CHUNK)], idxj_v.at[b], sem_in.at[b]),
                (idxi_hbm.at[pl.ds(base, CHUNK)], idxi_v.at[b], sem_in.at[b]),
                (g_hbm.at[pl.ds(gbase, CHUNK), :], g_v.at[b], sem_in.at[b]),
            )

        def in_start(ci, b):
            for args in in_copies(ci, b):
                pltpu.async_copy(*args)

        def in_wait(ci, b):
            for args in in_copies(ci, b):
                pltpu.make_async_copy(*args).wait()

        def gat_start(b):
            pltpu.async_copy(xj_hbm.at[idxj_v.at[b]], rows_v.at[b],
                             sem_gat.at[b])

        def gat_wait(b):
            pltpu.make_async_copy(xj_hbm.at[idxj_v.at[b]], rows_v.at[b],
                                  sem_gat.at[b]).wait()

        def sc_start(b):
            pltpu.async_copy(rows_v.at[b], acc_sh.at[idxi_v.at[b]],
                             sem_sc.at[b], add=True)

        def sc_wait(b):
            pltpu.make_async_copy(rows_v.at[b], acc_sh.at[idxi_v.at[b]],
                                  sem_sc.at[b]).wait()

        def mul(b):
            def row_body(i, c2):
                for d in range(D // 16):
                    s = pl.ds(d * 16, 16)
                    rows_v[b, i, s] = rows_v[b, i, s] * g_v[b, i, s]
                return c2

            lax.fori_loop(0, CHUNK, row_body, 0)

        def process(ci, b, guards_static):
            gat_wait(b)
            if (not guards_static) or ci + 1 < NCHUNK:
                b1 = (b + 1) % NB
                in_wait(ci + 1, b1)
                gat_start(b1)
            mul(b)
            sc_start(b)
            if (not guards_static) or ci + 3 < NCHUNK:
                b3 = (b + 3) % NB
                if (not guards_static) or ci + 3 - NB >= 0:
                    sc_wait(b3)
                in_start(ci + 3, b3)

        in_start(0, 0)
        in_start(1, 1)
        in_start(2, 2)
        in_wait(0, 0)
        gat_start(0)
        for ci in range(NB):
            process(ci, ci % NB, guards_static=True)

        def outer_body(o, carry):
            for b in range(NB):
                process(o * NB + b, b, guards_static=False)
            return carry

        lax.fori_loop(1, NCHUNK // NB - 1, outer_body, 0)

        for ci in range((NCHUNK // NB - 1) * NB, NCHUNK):
            process(ci, ci % NB, guards_static=True)
        for b in range(NB):
            sc_wait(b)

        plsc.subcore_barrier()
        pltpu.sync_copy(acc_sh.at[pl.ds(zbase, ROWS_PER_SUB), :],
                        out_hbm.at[cid, pl.ds(zbase, ROWS_PER_SUB), :])

    return edge_kernel


_edge_kernels = [_make_edge_kernel(h * (E // NHALF)) for h in range(NHALF)]


def _g_half(desc_half, W_perm):
    return pl.pallas_call(
        _matmul_body,
        grid=(E // NHALF // EDGE_BLK,),
        in_specs=[pl.BlockSpec((EDGE_BLK, R), lambda i: (i, 0)),
                  _full((R, D))],
        out_specs=pl.BlockSpec((EDGE_BLK, D), lambda i: (i, 0)),
        out_shape=jax.ShapeDtypeStruct((E // NHALF, D), jnp.float32),
    )(desc_half, W_perm)


def _node_stack_body(x_ref, p0_ref, p1_ref, p2_ref, p3_ref, wi_ref, bi_ref,
                     rw1_ref, rb1_ref, rw2_ref, rb2_ref,
                     wo_ref, bo_ref, sc_ref,
                     fw1_ref, fb1_ref, fw2_ref, fb2_ref, o_ref):
    x = x_ref[...]
    msg = (jnp.dot(x, wi_ref[...], preferred_element_type=jnp.float32)
           + bi_ref[...]
           + (p0_ref[...] + p1_ref[...]) + (p2_ref[...] + p3_ref[...]))
    for k in range(rw1_ref.shape[0]):
        h = jnp.dot(msg, rw1_ref[k], preferred_element_type=jnp.float32)
        h = h + rb1_ref[k]
        msg = msg + jnp.dot(h, rw2_ref[k],
                            preferred_element_type=jnp.float32) + rb2_ref[k]
    y = (sc_ref[...] * x
         + jnp.dot(msg, wo_ref[...], preferred_element_type=jnp.float32)
         + bo_ref[...])
    for k in range(fw1_ref.shape[0]):
        h = jnp.dot(y, fw1_ref[k], preferred_element_type=jnp.float32)
        h = h + fb1_ref[k]
        y = y + jnp.dot(h, fw2_ref[k],
                        preferred_element_type=jnp.float32) + fb2_ref[k]
    o_ref[...] = y


def _full(shape):
    return pl.BlockSpec(shape, lambda i: tuple(0 for _ in shape))


def kernel(features, descriptors, idx_i, idx_j, W_r2a, W_i, b_i, W_j, b_j,
           res_ij_W1, res_ij_b1, res_ij_W2, res_ij_b2, W_out, b_out, scaling,
           res_f_W1, res_f_b1, res_f_W2, res_f_b2):
    idx_i = idx_i.astype(jnp.int32)
    idx_j = idx_j.astype(jnp.int32)

    xj_all = pl.pallas_call(
        _dense_body,
        grid=(N // NODE_BLK,),
        in_specs=[pl.BlockSpec((NODE_BLK, D), lambda i: (i, 0)),
                  _full((D, D)),
                  _full((1, D))],
        out_specs=pl.BlockSpec((NODE_BLK, D), lambda i: (i, 0)),
        out_shape=jax.ShapeDtypeStruct((N, D), jnp.float32),
    )(features, W_j, b_j.reshape(1, D))

    zeros = jnp.zeros((NPAD, D), jnp.float32)
    EH = E // NHALF
    parts = []
    for h in range(NHALF):
        g_h = _g_half(descriptors[h * EH:(h + 1) * EH], W_r2a)
        p_h = _edge_kernels[h](g_h, idx_j, idx_i, xj_all, zeros)
        parts.append(p_h[:, :N, :])

    out = pl.pallas_call(
        _node_stack_body,
        grid=(N // NODE_BLK,),
        in_specs=[pl.BlockSpec((NODE_BLK, D), lambda i: (i, 0)),
                  pl.BlockSpec((NODE_BLK, D), lambda i: (i, 0)),
                  pl.BlockSpec((NODE_BLK, D), lambda i: (i, 0)),
                  pl.BlockSpec((NODE_BLK, D), lambda i: (i, 0)),
                  pl.BlockSpec((NODE_BLK, D), lambda i: (i, 0)),
                  _full((D, D)), _full((1, D)),
                  _full((2, D, D)), _full((2, 1, D)),
                  _full((2, D, D)), _full((2, 1, D)),
                  _full((D, D)), _full((1, D)), _full((1, D)),
                  _full((2, D, D)), _full((2, 1, D)),
                  _full((2, D, D)), _full((2, 1, D))],
        out_specs=pl.BlockSpec((NODE_BLK, D), lambda i: (i, 0)),
        out_shape=jax.ShapeDtypeStruct((N, D), jnp.float32),
    )(features, parts[0][0], parts[0][1], parts[1][0], parts[1][1],
      W_i, b_i.reshape(1, D),
      res_ij_W1, res_ij_b1.reshape(2, 1, D),
      res_ij_W2, res_ij_b2.reshape(2, 1, D),
      W_out, b_out.reshape(1, D), scaling.reshape(1, D),
      res_f_W1, res_f_b1.reshape(2, 1, D),
      res_f_W2, res_f_b2.reshape(2, 1, D))
    return out
